# baseline (speedup 1.0000x reference)
import functools

import jax
import jax.numpy as jnp
from jax import lax
from jax.experimental import pallas as pl
from jax.experimental.pallas import tpu as pltpu
from jax.experimental.pallas import tpu_sc as plsc

N = 10000
E = 320000
D_IN = 128
D_HID = 128
D_OUT = 40
D_OUT_PAD = 128

NC = 2
NS = 16
NW = NC * NS
CHUNK = 128
CPW = 80
NBLK = 2
CPB = CPW // NBLK
CPW0 = 120
CPW1 = 40
K_WB = 10
EPW = CHUNK * CPW
EPAD = NW * EPW
NPAD = 10240
RPS = NPAD // NS
LANES = 16

_EPS = 1e-5


def _sc_mesh():
    return plsc.VectorSubcoreMesh(core_axis_name="c", subcore_axis_name="s")


@functools.partial(
    pl.kernel,
    out_type=jax.ShapeDtypeStruct((NC, NPAD, LANES), jnp.float32),
    mesh=_sc_mesh(),
    scratch_types=[
        pltpu.VMEM((CPW, CHUNK), jnp.int32),
        pltpu.VMEM((CHUNK, LANES), jnp.float32),
        pltpu.VMEM_SHARED((NPAD, LANES), jnp.float32),
    ],
)
def _sc_degree(dst_hbm, out_hbm, dst_v, buf_v, acc_sh):
    c = lax.axis_index("c")
    s = lax.axis_index("s")
    wid = s * NC + c

    @pl.loop(0, CHUNK)
    def _(i):
        buf_v[i, pl.ds(0, LANES)] = jnp.zeros((LANES,), jnp.float32)

    @pl.loop(0, RPS, step=CHUNK)
    def _(r):
        pltpu.sync_copy(buf_v, acc_sh.at[pl.ds(s * RPS + r, CHUNK)])

    @pl.loop(0, CHUNK)
    def _(i):
        buf_v[i, pl.ds(0, LANES)] = jnp.ones((LANES,), jnp.float32)

    pltpu.sync_copy(dst_hbm.at[pl.ds(wid * CPW, CPW)], dst_v)
    plsc.subcore_barrier()

    @pl.loop(0, CPW)
    def _(j):
        pltpu.sync_copy(buf_v, acc_sh.at[dst_v.at[j]], add=True)

    plsc.subcore_barrier()
    pltpu.sync_copy(acc_sh.at[pl.ds(s * RPS, RPS)],
                    out_hbm.at[c, pl.ds(s * RPS, RPS)])


def _make_edge_pass(d):
    @functools.partial(
        pl.kernel,
        out_type=jax.ShapeDtypeStruct((NC, NPAD, d), jnp.float32),
        mesh=_sc_mesh(),
        scratch_types=[
            pltpu.VMEM((CPB, CHUNK), jnp.int32),
            pltpu.VMEM((CPB, CHUNK), jnp.int32),
            pltpu.VMEM((CHUNK, d), jnp.float32),
            pltpu.VMEM((CHUNK, d), jnp.float32),
            pltpu.VMEM_SHARED((NPAD, d), jnp.float32),
            pltpu.SemaphoreType.DMA,
            pltpu.SemaphoreType.DMA,
            pltpu.SemaphoreType.DMA,
        ],
    )
    def edge_pass(g_hbm, src_hbm, dst_hbm, out_hbm, src_v, dst_v,
                  rows0_v, rows1_v, acc_sh, sem0, sem1, semw):
        c = lax.axis_index("c")
        s = lax.axis_index("s")

        with jax.named_scope("ep_zero"):
            @pl.loop(0, CHUNK)
            def _(i):
                @pl.loop(0, d, step=LANES)
                def _(j):
                    rows0_v[i, pl.ds(j, LANES)] = jnp.zeros((LANES,), jnp.float32)

            @pl.loop(0, RPS, step=CHUNK)
            def _(r):
                pltpu.sync_copy(rows0_v, acc_sh.at[pl.ds(s * RPS + r, CHUNK)])

            plsc.subcore_barrier()

        n_chunks = jnp.where(c == 0, CPW0, CPW1)
        base0 = jnp.where(c == 0, s * CPW0, NS * CPW0 + s * CPW1)

        with jax.named_scope("ep_loop"):
            @pl.loop(0, n_chunks, step=CPB)
            def _(p):
                base = base0 + p
                pltpu.sync_copy(src_hbm.at[pl.ds(base, CPB)], src_v)
                pltpu.sync_copy(dst_hbm.at[pl.ds(base, CPB)], dst_v)

                @pl.loop(0, CPB, step=2)
                def _(j):
                    d0 = pltpu.async_copy(g_hbm.at[src_v.at[j]], rows0_v, sem0)
                    d1 = pltpu.async_copy(g_hbm.at[src_v.at[j + 1]], rows1_v, sem1)
                    d0.wait()
                    pltpu.sync_copy(rows0_v, acc_sh.at[dst_v.at[j]], add=True)
                    d1.wait()
                    pltpu.sync_copy(rows1_v, acc_sh.at[dst_v.at[j + 1]], add=True)

        with jax.named_scope("ep_flush"):
            plsc.subcore_barrier()
            descs = []
            for k in range(K_WB):
                off = s * RPS + k * (RPS // K_WB)
                descs.append(pltpu.async_copy(
                    acc_sh.at[pl.ds(off, RPS // K_WB)],
                    out_hbm.at[c, pl.ds(off, RPS // K_WB)], semw))
            for dsc in descs:
                dsc.wait()

    return edge_pass


_edge_pass_128 = _make_edge_pass(D_HID)


def _matmul_t(a, w):
    return lax.dot_general(a, w, (((1,), (1,)), ((), ())),
                           precision=lax.Precision.HIGHEST)


def _pair_norm(t):
    t = t - jnp.mean(t, axis=0, keepdims=True)
    return t * lax.rsqrt(_EPS + jnp.sum(t * t) / N)


def _tc_h0_body(x_ref, w0_ref, h0_ref):
    h0_ref[...] = _matmul_t(x_ref[...], w0_ref[...])


def _tc_h0(x, w0):
    return pl.pallas_call(
        _tc_h0_body,
        out_shape=jax.ShapeDtypeStruct((N, D_HID), jnp.float32),
    )(x, w0)


def _tc_prep_body(degp_ref, h0_ref, dis_ref, g0_ref):
    deg = degp_ref[0, :N, 0:1] + degp_ref[1, :N, 0:1] + 1.0
    dis = lax.rsqrt(deg)
    dis_ref[...] = dis
    g0_ref[...] = dis * h0_ref[...]


def _tc_prep(degp, h0):
    return pl.pallas_call(
        _tc_prep_body,
        out_shape=[
            jax.ShapeDtypeStruct((N, 1), jnp.float32),
            jax.ShapeDtypeStruct((N, D_HID), jnp.float32),
        ],
    )(degp, h0)


def _tc_mid_body(acc_ref, g_ref, dis_ref, b_ref, wn_ref, gn_ref):
    t = acc_ref[0, :N, :] + acc_ref[1, :N, :] + g_ref[...]
    t = dis_ref[...] * t + b_ref[...]
    t = _pair_norm(t)
    t = jnp.maximum(t, 0.0)
    gn_ref[...] = dis_ref[...] * _matmul_t(t, wn_ref[...])


def _tc_mid(acc, g, dis, b, wn, d_next):
    return pl.pallas_call(
        _tc_mid_body,
        out_shape=jax.ShapeDtypeStruct((N, d_next), jnp.float32),
    )(acc, g, dis, b, wn)


def _tc_final_body(acc_ref, g_ref, dis_ref, b_ref, out_ref):
    t = acc_ref[0, :N, :] + acc_ref[1, :N, :] + g_ref[...]
    t = dis_ref[...] * t + b_ref[...]
    out_ref[...] = _pair_norm(t)


def _tc_final(acc, g, dis, b):
    return pl.pallas_call(
        _tc_final_body,
        out_shape=jax.ShapeDtypeStruct((N, D_OUT_PAD), jnp.float32),
    )(acc, g, dis, b)


def kernel(x, edge_index, W0, b0, W1, b1, W2, b2):
    assert x.shape == (N, D_IN) and edge_index.shape == (2, E)

    src = edge_index[0].astype(jnp.int32)
    dst = edge_index[1].astype(jnp.int32)
    pad = EPAD - E
    src_p = jnp.concatenate([src, jnp.zeros((pad,), jnp.int32)])
    dst_p = jnp.concatenate([dst, jnp.full((pad,), N, jnp.int32)])
    src_p = src_p.reshape(NW * CPW, CHUNK)
    dst_p = dst_p.reshape(NW * CPW, CHUNK)

    W2p = jnp.zeros((D_OUT_PAD, D_HID), jnp.float32).at[:D_OUT].set(W2)
    b2p = jnp.zeros((1, D_OUT_PAD), jnp.float32).at[0, :D_OUT].set(b2)

    degp = _sc_degree(dst_p)
    h0 = _tc_h0(x, W0)
    dis, g0 = _tc_prep(degp, h0)

    acc0 = _edge_pass_128(g0, src_p, dst_p)
    g1 = _tc_mid(acc0, g0, dis, b0.reshape(1, D_HID), W1, D_HID)

    acc1 = _edge_pass_128(g1, src_p, dst_p)
    g2 = _tc_mid(acc1, g1, dis, b1.reshape(1, D_HID), W2p, D_OUT_PAD)

    acc2 = _edge_pass_128(g2, src_p, dst_p)
    out = _tc_final(acc2, g2, dis, b2p)

    return out[:, :D_OUT]


# speedup vs baseline: 2.5907x; 2.5907x over previous
import functools

import jax
import jax.numpy as jnp
from jax import lax
from jax.experimental import pallas as pl
from jax.experimental.pallas import tpu as pltpu
from jax.experimental.pallas import tpu_sc as plsc

N = 10000
E = 320000
D_IN = 128
D_HID = 128
D_OUT = 40
D_OUT_PAD = 128

NC = 2
NS = 16
NW = NC * NS
CHUNK = 128
CPW = 80
NBLK = 2
CPB = CPW // NBLK
CPW0 = 80
CPW1 = 80
K_WB = 10
EPW = CHUNK * CPW
EPAD = NW * EPW
NPAD = 10240
RPS = NPAD // NS
LANES = 16

_EPS = 1e-5


def _sc_mesh():
    return plsc.VectorSubcoreMesh(core_axis_name="c", subcore_axis_name="s")


@functools.partial(
    pl.kernel,
    out_type=jax.ShapeDtypeStruct((NC, NPAD, LANES), jnp.float32),
    mesh=_sc_mesh(),
    scratch_types=[
        pltpu.VMEM((CPW, CHUNK), jnp.int32),
        pltpu.VMEM((CHUNK, LANES), jnp.float32),
        pltpu.VMEM_SHARED((NPAD, LANES), jnp.float32),
    ],
)
def _sc_degree(dst_hbm, out_hbm, dst_v, buf_v, acc_sh):
    c = lax.axis_index("c")
    s = lax.axis_index("s")
    wid = s * NC + c

    @pl.loop(0, CHUNK)
    def _(i):
        buf_v[i, pl.ds(0, LANES)] = jnp.zeros((LANES,), jnp.float32)

    @pl.loop(0, RPS, step=CHUNK)
    def _(r):
        pltpu.sync_copy(buf_v, acc_sh.at[pl.ds(s * RPS + r, CHUNK)])

    @pl.loop(0, CHUNK)
    def _(i):
        buf_v[i, pl.ds(0, LANES)] = jnp.ones((LANES,), jnp.float32)

    pltpu.sync_copy(dst_hbm.at[pl.ds(wid * CPW, CPW)], dst_v)
    plsc.subcore_barrier()

    @pl.loop(0, CPW)
    def _(j):
        pltpu.sync_copy(buf_v, acc_sh.at[dst_v.at[j]], add=True)

    plsc.subcore_barrier()
    pltpu.sync_copy(acc_sh.at[pl.ds(s * RPS, RPS)],
                    out_hbm.at[c, pl.ds(s * RPS, RPS)])


def _make_edge_pass(d):
    @functools.partial(
        pl.kernel,
        out_type=jax.ShapeDtypeStruct((NC, NPAD, d), jnp.float32),
        mesh=_sc_mesh(),
        scratch_types=[
            pltpu.VMEM((CPB, CHUNK), jnp.int32),
            pltpu.VMEM((CPB, CHUNK), jnp.int32),
            pltpu.VMEM((CHUNK, d), jnp.float32),
            pltpu.VMEM((CHUNK, d), jnp.float32),
            pltpu.VMEM_SHARED((NPAD, d), jnp.float32),
            pltpu.SemaphoreType.DMA,
            pltpu.SemaphoreType.DMA,
            pltpu.SemaphoreType.DMA,
        ],
    )
    def edge_pass(g_hbm, src_hbm, dst_hbm, out_hbm, src_v, dst_v,
                  rows0_v, rows1_v, acc_sh, sem0, sem1, semw):
        c = lax.axis_index("c")
        s = lax.axis_index("s")

        with jax.named_scope("ep_zero"):
            @pl.loop(0, CHUNK)
            def _(i):
                @pl.loop(0, d, step=LANES)
                def _(j):
                    rows0_v[i, pl.ds(j, LANES)] = jnp.zeros((LANES,), jnp.float32)

            @pl.loop(0, RPS, step=CHUNK)
            def _(r):
                pltpu.sync_copy(rows0_v, acc_sh.at[pl.ds(s * RPS + r, CHUNK)])

            plsc.subcore_barrier()

        n_chunks = jnp.where(c == 0, CPW0, CPW1)
        base0 = jnp.where(c == 0, s * CPW0, NS * CPW0 + s * CPW1)

        with jax.named_scope("ep_loop"):
            @pl.loop(0, n_chunks, step=CPB)
            def _(p):
                base = base0 + p
                pltpu.sync_copy(src_hbm.at[pl.ds(base, CPB)], src_v)
                pltpu.sync_copy(dst_hbm.at[pl.ds(base, CPB)], dst_v)

                @pl.loop(0, CPB, step=2)
                def _(j):
                    d0 = pltpu.async_copy(g_hbm.at[src_v.at[j]], rows0_v, sem0)
                    d1 = pltpu.async_copy(g_hbm.at[src_v.at[j + 1]], rows1_v, sem1)
                    d0.wait()
                    pltpu.sync_copy(rows0_v, acc_sh.at[dst_v.at[j]], add=True)
                    d1.wait()
                    pltpu.sync_copy(rows1_v, acc_sh.at[dst_v.at[j + 1]], add=True)

        with jax.named_scope("ep_flush"):
            plsc.subcore_barrier()
            descs = []
            for k in range(K_WB):
                off = s * RPS + k * (RPS // K_WB)
                descs.append(pltpu.async_copy(
                    acc_sh.at[pl.ds(off, RPS // K_WB)],
                    out_hbm.at[c, pl.ds(off, RPS // K_WB)], semw))
            for dsc in descs:
                dsc.wait()

    return edge_pass


_edge_pass_128 = _make_edge_pass(D_HID)


def _matmul_t(a, w):
    return lax.dot_general(a, w, (((1,), (1,)), ((), ())),
                           precision=lax.Precision.HIGHEST)


def _pair_norm(t):
    t = t - jnp.mean(t, axis=0, keepdims=True)
    return t * lax.rsqrt(_EPS + jnp.sum(t * t) / N)


def _tc_h0_body(x_ref, w0_ref, h0_ref):
    h0_ref[...] = _matmul_t(x_ref[...], w0_ref[...])


def _tc_h0(x, w0):
    return pl.pallas_call(
        _tc_h0_body,
        out_shape=jax.ShapeDtypeStruct((N, D_HID), jnp.float32),
    )(x, w0)


def _tc_prep_body(degp_ref, h0_ref, dis_ref, g0_ref):
    deg = degp_ref[0, :N, 0:1] + degp_ref[1, :N, 0:1] + 1.0
    dis = lax.rsqrt(deg)
    dis_ref[...] = dis
    g0_ref[...] = dis * h0_ref[...]


def _tc_prep(degp, h0):
    return pl.pallas_call(
        _tc_prep_body,
        out_shape=[
            jax.ShapeDtypeStruct((N, 1), jnp.float32),
            jax.ShapeDtypeStruct((N, D_HID), jnp.float32),
        ],
    )(degp, h0)


def _tc_mid_body(acc_ref, g_ref, dis_ref, b_ref, wn_ref, gn_ref):
    t = acc_ref[0, :N, :] + acc_ref[1, :N, :] + g_ref[...]
    t = dis_ref[...] * t + b_ref[...]
    t = _pair_norm(t)
    t = jnp.maximum(t, 0.0)
    gn_ref[...] = dis_ref[...] * _matmul_t(t, wn_ref[...])


def _tc_mid(acc, g, dis, b, wn, d_next):
    return pl.pallas_call(
        _tc_mid_body,
        out_shape=jax.ShapeDtypeStruct((N, d_next), jnp.float32),
    )(acc, g, dis, b, wn)


def _tc_final_body(acc_ref, g_ref, dis_ref, b_ref, out_ref):
    t = acc_ref[0, :N, :] + acc_ref[1, :N, :] + g_ref[...]
    t = dis_ref[...] * t + b_ref[...]
    out_ref[...] = _pair_norm(t)


def _tc_final(acc, g, dis, b):
    return pl.pallas_call(
        _tc_final_body,
        out_shape=jax.ShapeDtypeStruct((N, D_OUT_PAD), jnp.float32),
    )(acc, g, dis, b)


def kernel(x, edge_index, W0, b0, W1, b1, W2, b2):
    assert x.shape == (N, D_IN) and edge_index.shape == (2, E)

    src = edge_index[0].astype(jnp.int32)
    dst = edge_index[1].astype(jnp.int32)
    pad = EPAD - E
    pad_i = jnp.arange(pad, dtype=jnp.int32)
    src_p = jnp.concatenate([src, pad_i % N])
    dst_p = jnp.concatenate([dst, N + pad_i % (NPAD - N)])
    src_p = src_p.reshape(NW * CPW, CHUNK)
    dst_p = dst_p.reshape(NW * CPW, CHUNK)

    W2p = jnp.zeros((D_OUT_PAD, D_HID), jnp.float32).at[:D_OUT].set(W2)
    b2p = jnp.zeros((1, D_OUT_PAD), jnp.float32).at[0, :D_OUT].set(b2)

    degp = _sc_degree(dst_p)
    h0 = _tc_h0(x, W0)
    dis, g0 = _tc_prep(degp, h0)

    acc0 = _edge_pass_128(g0, src_p, dst_p)
    g1 = _tc_mid(acc0, g0, dis, b0.reshape(1, D_HID), W1, D_HID)

    acc1 = _edge_pass_128(g1, src_p, dst_p)
    g2 = _tc_mid(acc1, g1, dis, b1.reshape(1, D_HID), W2p, D_OUT_PAD)

    acc2 = _edge_pass_128(g2, src_p, dst_p)
    out = _tc_final(acc2, g2, dis, b2p)

    return out[:, :D_OUT]


# speedup vs baseline: 2.6252x; 1.0133x over previous
import functools

import jax
import jax.numpy as jnp
from jax import lax
from jax.experimental import pallas as pl
from jax.experimental.pallas import tpu as pltpu
from jax.experimental.pallas import tpu_sc as plsc

N = 10000
E = 320000
D_IN = 128
D_HID = 128
D_OUT = 40
D_OUT_PAD = 128

NC = 2
NS = 16
NW = NC * NS
CHUNK = 128
CPW = 80
NBLK = 2
CPB = CPW // NBLK
CPW0 = 80
CPW1 = 80
K_WB = 10
EPW = CHUNK * CPW
EPAD = NW * EPW
NPAD = 10240
RPS = NPAD // NS
LANES = 16

_EPS = 1e-5


def _sc_mesh():
    return plsc.VectorSubcoreMesh(core_axis_name="c", subcore_axis_name="s")


@functools.partial(
    pl.kernel,
    out_type=jax.ShapeDtypeStruct((NC, NPAD, LANES), jnp.float32),
    mesh=_sc_mesh(),
    scratch_types=[
        pltpu.VMEM((CPW, CHUNK), jnp.int32),
        pltpu.VMEM((CHUNK, LANES), jnp.float32),
        pltpu.VMEM_SHARED((NPAD, LANES), jnp.float32),
    ],
)
def _sc_degree(dst_hbm, out_hbm, dst_v, buf_v, acc_sh):
    c = lax.axis_index("c")
    s = lax.axis_index("s")
    wid = s * NC + c

    @pl.loop(0, CHUNK)
    def _(i):
        buf_v[i, pl.ds(0, LANES)] = jnp.zeros((LANES,), jnp.float32)

    @pl.loop(0, RPS, step=CHUNK)
    def _(r):
        pltpu.sync_copy(buf_v, acc_sh.at[pl.ds(s * RPS + r, CHUNK)])

    @pl.loop(0, CHUNK)
    def _(i):
        buf_v[i, pl.ds(0, LANES)] = jnp.ones((LANES,), jnp.float32)

    pltpu.sync_copy(dst_hbm.at[pl.ds(wid * CPW, CPW)], dst_v)
    plsc.subcore_barrier()

    @pl.loop(0, CPW)
    def _(j):
        pltpu.sync_copy(buf_v, acc_sh.at[dst_v.at[j]], add=True)

    plsc.subcore_barrier()
    pltpu.sync_copy(acc_sh.at[pl.ds(s * RPS, RPS)],
                    out_hbm.at[c, pl.ds(s * RPS, RPS)])


def _make_edge_pass(d):
    @functools.partial(
        pl.kernel,
        out_type=jax.ShapeDtypeStruct((NC, NPAD, d), jnp.float32),
        mesh=_sc_mesh(),
        scratch_types=[
            pltpu.VMEM((CPB, CHUNK), jnp.int32),
            pltpu.VMEM((CPB, CHUNK), jnp.int32),
            pltpu.VMEM((CHUNK, d), jnp.float32),
            pltpu.VMEM((CHUNK, d), jnp.float32),
            pltpu.VMEM_SHARED((NPAD, d), jnp.float32),
            pltpu.SemaphoreType.DMA,
            pltpu.SemaphoreType.DMA,
            pltpu.SemaphoreType.DMA,
            pltpu.SemaphoreType.DMA,
            pltpu.SemaphoreType.DMA,
        ],
    )
    def edge_pass(g_hbm, src_hbm, dst_hbm, out_hbm, src_v, dst_v,
                  rows0_v, rows1_v, acc_sh, sem0, sem1, semw, semS0, semS1):
        c = lax.axis_index("c")
        s = lax.axis_index("s")

        with jax.named_scope("ep_zero"):
            @pl.loop(0, CHUNK)
            def _(i):
                @pl.loop(0, d, step=LANES)
                def _(j):
                    rows0_v[i, pl.ds(j, LANES)] = jnp.zeros((LANES,), jnp.float32)

            @pl.loop(0, RPS, step=CHUNK)
            def _(r):
                pltpu.sync_copy(rows0_v, acc_sh.at[pl.ds(s * RPS + r, CHUNK)])

            plsc.subcore_barrier()

        n_chunks = jnp.where(c == 0, CPW0, CPW1)
        base0 = jnp.where(c == 0, s * CPW0, NS * CPW0 + s * CPW1)

        with jax.named_scope("ep_loop"):
            @pl.loop(0, n_chunks, step=CPB)
            def _(p):
                base = base0 + p
                pltpu.sync_copy(src_hbm.at[pl.ds(base, CPB)], src_v)
                pltpu.sync_copy(dst_hbm.at[pl.ds(base, CPB)], dst_v)

                @pl.loop(0, CPB, step=2)
                def _(j):
                    d0 = pltpu.async_copy(g_hbm.at[src_v.at[j]], rows0_v, sem0)
                    d1 = pltpu.async_copy(g_hbm.at[src_v.at[j + 1]], rows1_v, sem1)
                    d0.wait()
                    s0 = pltpu.async_copy(rows0_v, acc_sh.at[dst_v.at[j]],
                                          semS0, add=True)
                    d1.wait()
                    s1 = pltpu.async_copy(rows1_v, acc_sh.at[dst_v.at[j + 1]],
                                          semS1, add=True)
                    s0.wait()
                    s1.wait()

        with jax.named_scope("ep_flush"):
            plsc.subcore_barrier()
            descs = []
            for k in range(K_WB):
                off = s * RPS + k * (RPS // K_WB)
                descs.append(pltpu.async_copy(
                    acc_sh.at[pl.ds(off, RPS // K_WB)],
                    out_hbm.at[c, pl.ds(off, RPS // K_WB)], semw))
            for dsc in descs:
                dsc.wait()

    return edge_pass


_edge_pass_128 = _make_edge_pass(D_HID)


def _matmul_t(a, w):
    return lax.dot_general(a, w, (((1,), (1,)), ((), ())),
                           precision=lax.Precision.HIGHEST)


def _pair_norm(t):
    t = t - jnp.mean(t, axis=0, keepdims=True)
    return t * lax.rsqrt(_EPS + jnp.sum(t * t) / N)


def _tc_h0_body(x_ref, w0_ref, h0_ref):
    h0_ref[...] = _matmul_t(x_ref[...], w0_ref[...])


def _tc_h0(x, w0):
    return pl.pallas_call(
        _tc_h0_body,
        out_shape=jax.ShapeDtypeStruct((N, D_HID), jnp.float32),
    )(x, w0)


def _tc_prep_body(degp_ref, h0_ref, dis_ref, g0_ref):
    deg = degp_ref[0, :N, 0:1] + degp_ref[1, :N, 0:1] + 1.0
    dis = lax.rsqrt(deg)
    dis_ref[...] = dis
    g0_ref[...] = dis * h0_ref[...]


def _tc_prep(degp, h0):
    return pl.pallas_call(
        _tc_prep_body,
        out_shape=[
            jax.ShapeDtypeStruct((N, 1), jnp.float32),
            jax.ShapeDtypeStruct((N, D_HID), jnp.float32),
        ],
    )(degp, h0)


def _tc_mid_body(acc_ref, g_ref, dis_ref, b_ref, wn_ref, gn_ref):
    t = acc_ref[0, :N, :] + acc_ref[1, :N, :] + g_ref[...]
    t = dis_ref[...] * t + b_ref[...]
    t = _pair_norm(t)
    t = jnp.maximum(t, 0.0)
    gn_ref[...] = dis_ref[...] * _matmul_t(t, wn_ref[...])


def _tc_mid(acc, g, dis, b, wn, d_next):
    return pl.pallas_call(
        _tc_mid_body,
        out_shape=jax.ShapeDtypeStruct((N, d_next), jnp.float32),
    )(acc, g, dis, b, wn)


def _tc_final_body(acc_ref, g_ref, dis_ref, b_ref, out_ref):
    t = acc_ref[0, :N, :] + acc_ref[1, :N, :] + g_ref[...]
    t = dis_ref[...] * t + b_ref[...]
    out_ref[...] = _pair_norm(t)


def _tc_final(acc, g, dis, b):
    return pl.pallas_call(
        _tc_final_body,
        out_shape=jax.ShapeDtypeStruct((N, D_OUT_PAD), jnp.float32),
    )(acc, g, dis, b)


def kernel(x, edge_index, W0, b0, W1, b1, W2, b2):
    assert x.shape == (N, D_IN) and edge_index.shape == (2, E)

    src = edge_index[0].astype(jnp.int32)
    dst = edge_index[1].astype(jnp.int32)
    pad = EPAD - E
    pad_i = jnp.arange(pad, dtype=jnp.int32)
    src_p = jnp.concatenate([src, pad_i % N])
    dst_p = jnp.concatenate([dst, N + pad_i % (NPAD - N)])
    src_p = src_p.reshape(NW * CPW, CHUNK)
    dst_p = dst_p.reshape(NW * CPW, CHUNK)

    W2p = jnp.zeros((D_OUT_PAD, D_HID), jnp.float32).at[:D_OUT].set(W2)
    b2p = jnp.zeros((1, D_OUT_PAD), jnp.float32).at[0, :D_OUT].set(b2)

    degp = _sc_degree(dst_p)
    h0 = _tc_h0(x, W0)
    dis, g0 = _tc_prep(degp, h0)

    acc0 = _edge_pass_128(g0, src_p, dst_p)
    g1 = _tc_mid(acc0, g0, dis, b0.reshape(1, D_HID), W1, D_HID)

    acc1 = _edge_pass_128(g1, src_p, dst_p)
    g2 = _tc_mid(acc1, g1, dis, b1.reshape(1, D_HID), W2p, D_OUT_PAD)

    acc2 = _edge_pass_128(g2, src_p, dst_p)
    out = _tc_final(acc2, g2, dis, b2p)

    return out[:, :D_OUT]
